# baseline (speedup 1.0000x reference)
import functools

import jax
import jax.numpy as jnp
from jax import lax
from jax.experimental import pallas as pl
from jax.experimental.pallas import tpu as pltpu
from jax.experimental.pallas import tpu_sc as plsc

_K = 16
_EPS = 1e-5


def _topk_body(c_ref, ct_ref, xxr_ref, idx_ref):
    b = pl.program_id(0)
    c = c_ref[0]
    ct = ct_ref[0]
    xxr = xxr_ref[0]
    inner = -2.0 * jnp.dot(c, ct, preferred_element_type=jnp.float32)
    pd = -xxr - inner
    G = pd.shape[1]
    col = jax.lax.broadcasted_iota(jnp.int32, pd.shape, 1)
    for t in range(_K):
        rowmax = jnp.max(pd, axis=1, keepdims=True)
        ismax = pd == rowmax
        arg = jnp.min(jnp.where(ismax, col, G), axis=1, keepdims=True)
        idx_ref[0, :, t] = arg[:, 0] + b * G
        pd = jnp.where(col == arg, float("-inf"), pd)


def _topk(center):
    B, G, _ = center.shape
    ct = jnp.transpose(center, (0, 2, 1))
    xx = jnp.sum(ct ** 2, axis=1, keepdims=True)
    return pl.pallas_call(
        _topk_body,
        grid=(B,),
        in_specs=[
            pl.BlockSpec((1, G, 3), lambda b: (b, 0, 0)),
            pl.BlockSpec((1, 3, G), lambda b: (b, 0, 0)),
            pl.BlockSpec((1, 1, G), lambda b: (b, 0, 0)),
        ],
        out_specs=pl.BlockSpec((1, G, _K), lambda b: (b, 0, 0)),
        out_shape=jax.ShapeDtypeStruct((B, G, _K), jnp.int32),
    )(center, ct, xx)


def _yz_body(x_ref, wa_ref, wd_ref, y_ref, z_ref):
    xb = x_ref[...]
    y_ref[...] = jnp.dot(xb, wa_ref[...], preferred_element_type=jnp.float32,
                         precision=jax.lax.Precision.HIGHEST)
    z_ref[...] = jnp.dot(xb, wd_ref[...], preferred_element_type=jnp.float32,
                         precision=jax.lax.Precision.HIGHEST)


def _yz(x2d, WaT, WdT, nblk):
    R, d = x2d.shape
    rb = R // nblk
    return pl.pallas_call(
        _yz_body,
        grid=(nblk,),
        in_specs=[
            pl.BlockSpec((rb, d), lambda i: (i, 0)),
            pl.BlockSpec((d, d), lambda i: (0, 0)),
            pl.BlockSpec((d, d), lambda i: (0, 0)),
        ],
        out_specs=[
            pl.BlockSpec((rb, d), lambda i: (i, 0)),
            pl.BlockSpec((rb, d), lambda i: (i, 0)),
        ],
        out_shape=[
            jax.ShapeDtypeStruct((R, d), jnp.float32),
            jax.ShapeDtypeStruct((R, d), jnp.float32),
        ],
    )(x2d, WaT, WdT)


_NW = 32
_PPW = 128
_CH = 8
_NCH = _PPW // _CH


def _sc_body(y_hbm, z_hbm, idx_hbm, m_hbm, n_hbm, p1_hbm, p2_hbm,
             idx_v, rows_v, z_v, m_v, n_v, s1_v, s2_v,
             gsem0, gsem1, osem0, osem1):
    d = 384
    nl = d // 16
    wid = lax.axis_index("s") * 2 + lax.axis_index("c")
    base = wid * _PPW

    pltpu.sync_copy(idx_hbm.at[pl.ds(base * _K, _PPW * _K)], idx_v)

    zero16 = jnp.zeros((16,), jnp.float32)
    for cc in range(nl):
        s1_v[pl.ds(cc * 16, 16)] = zero16
        s2_v[pl.ds(cc * 16, 16)] = zero16

    gsems = (gsem0, gsem1)
    osems = (osem0, osem1)

    def _issue_gather(ch, buf):
        pltpu.async_copy(
            y_hbm.at[idx_v.at[pl.ds(ch * (_CH * _K), _CH * _K)]],
            rows_v.at[buf], gsems[buf])

    _issue_gather(0, 0)

    def outer(i, carry):
        for b in range(2):
            ch = 2 * i + b

            @pl.when(ch + 1 < _NCH)
            def _():
                _issue_gather(ch + 1, 1 - b)

            pltpu.sync_copy(z_hbm.at[pl.ds(base + ch * _CH, _CH)], z_v)

            pltpu.make_async_copy(
                y_hbm.at[pl.ds(0, _CH * _K)], rows_v.at[b], gsems[b]).wait()

            @pl.when(i >= 1)
            def _():
                pltpu.make_async_copy(
                    y_hbm.at[pl.ds(0, _CH)], m_v.at[b], osems[b]).wait()
                pltpu.make_async_copy(
                    y_hbm.at[pl.ds(0, _CH)], n_v.at[b], osems[b]).wait()

            def point(p, c2):
                r0 = p * _K
                for cc in range(nl):
                    sl = pl.ds(cc * 16, 16)
                    v0 = rows_v[b, r0, sl]
                    macc = v0
                    nacc = v0
                    sacc = v0
                    qacc = v0 * v0
                    for j in range(1, _K):
                        v = rows_v[b, r0 + j, sl]
                        macc = jnp.maximum(macc, v)
                        nacc = jnp.minimum(nacc, v)
                        sacc = sacc + v
                        qacc = qacc + v * v
                    m_v[b, p, sl] = macc
                    n_v[b, p, sl] = nacc
                    zc = z_v[p, sl]
                    s1_v[sl] = s1_v[sl] + (sacc + 16.0 * zc)
                    s2_v[sl] = s2_v[sl] + (qacc + (2.0 * zc) * sacc
                                           + 16.0 * (zc * zc))
                return c2

            lax.fori_loop(0, _CH, point, 0)

            pltpu.async_copy(m_v.at[b], m_hbm.at[pl.ds(base + ch * _CH, _CH)],
                             osems[b])
            pltpu.async_copy(n_v.at[b], n_hbm.at[pl.ds(base + ch * _CH, _CH)],
                             osems[b])
        return carry

    lax.fori_loop(0, _NCH // 2, outer, 0)

    for b in range(2):
        pltpu.make_async_copy(y_hbm.at[pl.ds(0, _CH)], m_v.at[b],
                              osems[b]).wait()
        pltpu.make_async_copy(y_hbm.at[pl.ds(0, _CH)], n_v.at[b],
                              osems[b]).wait()

    pltpu.sync_copy(s1_v, p1_hbm.at[wid])
    pltpu.sync_copy(s2_v, p2_hbm.at[wid])


def _sc_gather_reduce(y2d, z2d, idx_flat):
    R, d = y2d.shape
    mesh = plsc.VectorSubcoreMesh(core_axis_name="c", subcore_axis_name="s",
                                  num_cores=2, num_subcores=16)
    fn = pl.kernel(
        _sc_body,
        out_type=[
            jax.ShapeDtypeStruct((R, d), jnp.float32),
            jax.ShapeDtypeStruct((R, d), jnp.float32),
            jax.ShapeDtypeStruct((_NW, d), jnp.float32),
            jax.ShapeDtypeStruct((_NW, d), jnp.float32),
        ],
        mesh=mesh,
        scratch_types=[
            pltpu.VMEM((_PPW * _K,), jnp.int32),
            pltpu.VMEM((2, _CH * _K, d), jnp.float32),
            pltpu.VMEM((_CH, d), jnp.float32),
            pltpu.VMEM((2, _CH, d), jnp.float32),
            pltpu.VMEM((2, _CH, d), jnp.float32),
            pltpu.VMEM((d,), jnp.float32),
            pltpu.VMEM((d,), jnp.float32),
            pltpu.SemaphoreType.DMA,
            pltpu.SemaphoreType.DMA,
            pltpu.SemaphoreType.DMA,
            pltpu.SemaphoreType.DMA,
        ],
    )
    return fn(y2d, z2d, idx_flat)


def _ep1_body(m_ref, n_ref, z_ref, p1_ref, p2_ref, g1_ref, b1_ref, w2t_ref,
              h2_ref, ps_ref, pq_ref, *, n1):
    mean1 = jnp.sum(p1_ref[...], axis=0, keepdims=True) / n1
    var1 = jnp.sum(p2_ref[...], axis=0, keepdims=True) / n1 - mean1 * mean1
    inv1 = jax.lax.rsqrt(var1 + _EPS)
    g1 = g1_ref[...]
    pooled = jnp.where(g1 >= 0, m_ref[...], n_ref[...]) + z_ref[...]
    h1 = (pooled - mean1) * (inv1 * g1) + b1_ref[...]
    h1 = jnp.where(h1 >= 0, h1, 0.2 * h1)
    h2 = jnp.dot(h1, w2t_ref[...], preferred_element_type=jnp.float32,
                 precision=jax.lax.Precision.HIGHEST)
    h2_ref[...] = h2
    ps_ref[0] = jnp.sum(h2, axis=0, keepdims=True)
    pq_ref[0] = jnp.sum(h2 * h2, axis=0, keepdims=True)


def _ep1(m2d, n2d, z2d, p1, p2, g1, b1, W2T, nblk):
    R, d = m2d.shape
    rb = R // nblk
    return pl.pallas_call(
        functools.partial(_ep1_body, n1=float(R * _K)),
        grid=(nblk,),
        in_specs=[
            pl.BlockSpec((rb, d), lambda i: (i, 0)),
            pl.BlockSpec((rb, d), lambda i: (i, 0)),
            pl.BlockSpec((rb, d), lambda i: (i, 0)),
            pl.BlockSpec((_NW, d), lambda i: (0, 0)),
            pl.BlockSpec((_NW, d), lambda i: (0, 0)),
            pl.BlockSpec((1, d), lambda i: (0, 0)),
            pl.BlockSpec((1, d), lambda i: (0, 0)),
            pl.BlockSpec((d, d), lambda i: (0, 0)),
        ],
        out_specs=[
            pl.BlockSpec((rb, d), lambda i: (i, 0)),
            pl.BlockSpec((1, 1, d), lambda i: (i, 0, 0)),
            pl.BlockSpec((1, 1, d), lambda i: (i, 0, 0)),
        ],
        out_shape=[
            jax.ShapeDtypeStruct((R, d), jnp.float32),
            jax.ShapeDtypeStruct((nblk, 1, d), jnp.float32),
            jax.ShapeDtypeStruct((nblk, 1, d), jnp.float32),
        ],
    )(m2d, n2d, z2d, p1, p2, g1, b1, W2T)


def _ep2_body(h2_ref, ps_ref, pq_ref, g2_ref, b2_ref, out_ref, *, n2):
    mean2 = jnp.sum(ps_ref[:, 0, :], axis=0, keepdims=True) / n2
    var2 = jnp.sum(pq_ref[:, 0, :], axis=0, keepdims=True) / n2 - mean2 * mean2
    inv2 = jax.lax.rsqrt(var2 + _EPS)
    out = (h2_ref[...] - mean2) * (inv2 * g2_ref[...]) + b2_ref[...]
    out_ref[...] = jnp.where(out >= 0, out, 0.2 * out)


def _ep2(h2, ps, pq, g2, b2, nblk):
    R, d = h2.shape
    rb = R // nblk
    return pl.pallas_call(
        functools.partial(_ep2_body, n2=float(R)),
        grid=(nblk,),
        in_specs=[
            pl.BlockSpec((rb, d), lambda i: (i, 0)),
            pl.BlockSpec((nblk, 1, d), lambda i: (0, 0, 0)),
            pl.BlockSpec((nblk, 1, d), lambda i: (0, 0, 0)),
            pl.BlockSpec((1, d), lambda i: (0, 0)),
            pl.BlockSpec((1, d), lambda i: (0, 0)),
        ],
        out_specs=pl.BlockSpec((rb, d), lambda i: (i, 0)),
        out_shape=jax.ShapeDtypeStruct((R, d), jnp.float32),
    )(h2, ps, pq, g2, b2)


def kernel(x, center, W1, gamma1, beta1, W2, gamma2, beta2):
    B, G, d = x.shape
    R = B * G
    nblk = 8

    idx = _topk(center)
    WaT = W1[:, :d].T
    WdT = (W1[:, d:] - W1[:, :d]).T
    x2d = x.reshape(R, d)
    y2d, z2d = _yz(x2d, WaT, WdT, nblk)

    idx_flat = idx.reshape(R * _K)
    m2d, n2d, p1, p2 = _sc_gather_reduce(y2d, z2d, idx_flat)

    g1 = gamma1.reshape(1, d)
    b1 = beta1.reshape(1, d)
    h2, ps, pq = _ep1(m2d, n2d, z2d, p1, p2, g1, b1, W2.T, nblk)
    out2d = _ep2(h2, ps, pq, gamma2.reshape(1, d), beta2.reshape(1, d), nblk)
    return out2d.reshape(B, G, d)
